# baseline (speedup 1.0000x reference)
import functools

import jax
import jax.numpy as jnp
from jax import lax
from jax.experimental import pallas as pl
from jax.experimental.pallas import tpu as pltpu
from jax.experimental.pallas import tpu_sc as plsc

DIM = 32
N_CLASSES = 1000
ZERO_ROW = N_CLASSES
N_PAD = 1008

_info = plsc.get_sparse_core_info()
NC, NS, L = _info.num_cores, _info.num_subcores, _info.num_lanes
NW = NC * NS

B = 4096
N = 200
NBT = B // 128
NDT = DIM // 8
OUT_ROWS = N * NDT * NBT * 8
REM = N % NW

_mesh = plsc.VectorSubcoreMesh(core_axis_name="c", subcore_axis_name="s")


@functools.partial(
    pl.kernel,
    out_type=jax.ShapeDtypeStruct((OUT_ROWS, 128), jnp.float32),
    mesh=_mesh,
    compiler_params=pltpu.CompilerParams(
        use_tc_tiling_on_sc=False, needs_layout_passes=False),
    scratch_types=[
        pltpu.VMEM((DIM, N_PAD), jnp.float32),
        pltpu.VMEM((NBT, 128), jnp.int32),
        pltpu.VMEM((NBT, 128), jnp.int32),
        pltpu.VMEM((NBT, 128), jnp.int32),
        pltpu.VMEM((NBT, 128), jnp.int32),
        pltpu.VMEM((NBT, 128), jnp.int32),
        pltpu.VMEM((128, 128), jnp.float32),
        pltpu.VMEM((128, 128), jnp.float32),
        pltpu.SemaphoreType.DMA,
        pltpu.SemaphoreType.DMA,
    ],
)
def _sc_embed(y_hbm, pad_hbm, tab_hbm, out_sup,
              tab_v, yv0, yv1, pad0, pad1, midx, rows_a, rows_b,
              sem_i, sem_w):
    wid = lax.axis_index("s") * NC + lax.axis_index("c")

    yvs = [yv0, yv1]
    pads = [pad0, pad1]
    rows_bufs = [rows_a, rows_b]

    NU = N * NDT // NW

    def issue_idx(j, par):
        n = (wid + j * NW) // NDT
        pltpu.async_copy(y_hbm.at[n], yvs[par], sem_i)
        pltpu.async_copy(pad_hbm.at[n], pads[par], sem_i)

    def wait_idx(par):
        pltpu.make_async_copy(y_hbm.at[0], yvs[par], sem_i).wait()
        pltpu.make_async_copy(pad_hbm.at[0], pads[par], sem_i).wait()

    issue_idx(0, 0)
    pltpu.sync_copy(tab_hbm, tab_v)

    def one_unit(j, par):
        u = wid + j * NW
        dt = u % NDT
        wait_idx(par)

        @pl.when(j + 1 < NU)
        def _prefetch():
            issue_idx(j + 1, 1 - par)

        yv, padv = yvs[par], pads[par]

        @pl.loop(0, NBT)
        def _mask(i):
            for v in range(8):
                y16 = yv[i, pl.ds(v * L, L)]
                p16 = padv[i, pl.ds(v * L, L)]
                midx[i, pl.ds(v * L, L)] = jnp.where(
                    p16 != 0, ZERO_ROW, y16)

        row0 = pl.multiple_of(u * (NBT * 8), 256)
        dsplats = [jnp.broadcast_to(dt * 8 + dr, (L,)) for dr in range(8)]
        for half in range(2):
            rows = rows_bufs[half]
            @pl.when(j > 0)
            def _drain():
                pltpu.make_async_copy(
                    rows, out_sup.at[pl.ds(0, 128)], sem_w).wait()

            @pl.loop(0, NBT // 2, unroll=4)
            def _bt(btl):
                bt = half * (NBT // 2) + btl
                m16s = [midx[bt, pl.ds(v * L, L)] for v in range(8)]
                for dr in range(8):
                    for v in range(8):
                        rows[btl * 8 + dr, pl.ds(v * L, L)] = (
                            plsc.load_gather(tab_v, [dsplats[dr], m16s[v]]))

            pltpu.async_copy(
                rows, out_sup.at[pl.ds(row0 + half * 128, 128)], sem_w)

    @pl.loop(0, NU // 2)
    def _pair(jp):
        for par in range(2):
            one_unit(jp * 2 + par, par)

    one_unit(NU - 1, 0)

    for rows in rows_bufs:
        pltpu.make_async_copy(rows, out_sup.at[pl.ds(0, 128)], sem_w).wait()


def _tc_query_body(qpat_ref, out_ref):
    out_ref[...] = qpat_ref[...]


_tc_query = pl.pallas_call(
    _tc_query_body,
    grid=(N,),
    in_specs=[pl.BlockSpec((NDT * NBT * 8, 128), lambda i: (0, 0))],
    out_specs=pl.BlockSpec((NDT * NBT * 8, 128), lambda i: (i, 0)),
    out_shape=jax.ShapeDtypeStruct((OUT_ROWS, 128), jnp.float32),
)


def kernel(y_support, padding_obs_support, n_obs_query, y_embedding, y_mask):
    yt = y_support.astype(jnp.int32).T.reshape(N, NBT, 128)
    padt = padding_obs_support.astype(jnp.int32).T.reshape(N, NBT, 128)
    tab_t = jnp.concatenate(
        [y_embedding,
         jnp.zeros((N_PAD - N_CLASSES, DIM), jnp.float32)], axis=0).T
    sup2d = _sc_embed(yt, padt, tab_t)
    qpat = jnp.broadcast_to(
        y_mask.reshape(NDT, 1, 8, 1), (NDT, NBT, 8, 128)).reshape(1024, 128)
    q2d = _tc_query(qpat)

    def to_out(x):
        x = x.reshape(N, NDT, NBT, 8, 128)
        x = x.transpose(2, 4, 0, 1, 3)
        return x.reshape(B, N, 1, DIM)

    return to_out(sup2d), to_out(q2d)


# speedup vs baseline: 1.0199x; 1.0199x over previous
import functools

import jax
import jax.numpy as jnp
from jax import lax
from jax.experimental import pallas as pl
from jax.experimental.pallas import tpu as pltpu
from jax.experimental.pallas import tpu_sc as plsc

DIM = 32
N_CLASSES = 1000
ZERO_ROW = N_CLASSES
N_PAD = 1008

_info = plsc.get_sparse_core_info()
NC, NS, L = _info.num_cores, _info.num_subcores, _info.num_lanes
NW = NC * NS

B = 4096
N = 200
NBT = B // 128
NDT = DIM // 8
OUT_ROWS = N * NDT * NBT * 8
REM = N % NW

_mesh = plsc.VectorSubcoreMesh(core_axis_name="c", subcore_axis_name="s")


@functools.partial(
    pl.kernel,
    out_type=jax.ShapeDtypeStruct((OUT_ROWS, 128), jnp.float32),
    mesh=_mesh,
    compiler_params=pltpu.CompilerParams(
        use_tc_tiling_on_sc=False, needs_layout_passes=False),
    scratch_types=[
        pltpu.VMEM((DIM, N_PAD), jnp.float32),
        pltpu.VMEM((NBT, 128), jnp.int32),
        pltpu.VMEM((NBT, 128), jnp.int32),
        pltpu.VMEM((NBT, 128), jnp.int32),
        pltpu.VMEM((NBT, 128), jnp.int32),
        pltpu.VMEM((NBT, 128), jnp.int32),
        pltpu.VMEM((128, 128), jnp.float32),
        pltpu.VMEM((128, 128), jnp.float32),
        pltpu.SemaphoreType.DMA,
        pltpu.SemaphoreType.DMA,
    ],
)
def _sc_embed(y_hbm, pad_hbm, tab_hbm, out_sup,
              tab_v, yv0, yv1, pad0, pad1, midx, rows_a, rows_b,
              sem_i, sem_w):
    wid = lax.axis_index("s") * NC + lax.axis_index("c")

    yvs = [yv0, yv1]
    pads = [pad0, pad1]
    rows_bufs = [rows_a, rows_b]

    NU = N * NDT // NW

    def issue_idx(j, par):
        n = (wid + j * NW) // NDT
        pltpu.async_copy(y_hbm.at[n], yvs[par], sem_i)
        pltpu.async_copy(pad_hbm.at[n], pads[par], sem_i)

    def wait_idx(par):
        pltpu.make_async_copy(y_hbm.at[0], yvs[par], sem_i).wait()
        pltpu.make_async_copy(pad_hbm.at[0], pads[par], sem_i).wait()

    issue_idx(0, 0)
    pltpu.sync_copy(tab_hbm, tab_v)

    def one_unit(j, par):
        u = wid + j * NW
        dt = u % NDT
        wait_idx(par)

        @pl.when(j + 1 < NU)
        def _prefetch():
            issue_idx(j + 1, 1 - par)

        yv, padv = yvs[par], pads[par]

        @pl.loop(0, NBT)
        def _mask(i):
            for v in range(8):
                y16 = yv[i, pl.ds(v * L, L)]
                p16 = padv[i, pl.ds(v * L, L)]
                midx[i, pl.ds(v * L, L)] = jnp.where(
                    p16 != 0, ZERO_ROW, y16)

        row0 = pl.multiple_of(u * (NBT * 8), 256)
        dsplats = [jnp.broadcast_to(dt * 8 + dr, (L,)) for dr in range(8)]
        for half in range(2):
            rows = rows_bufs[half]
            @pl.when(j > 0)
            def _drain():
                pltpu.make_async_copy(
                    rows, out_sup.at[pl.ds(0, 128)], sem_w).wait()

            @pl.loop(0, NBT // 2, unroll=2)
            def _bt(btl):
                bt = half * (NBT // 2) + btl
                m16s = [midx[bt, pl.ds(v * L, L)] for v in range(8)]
                for dr in range(8):
                    for v in range(8):
                        rows[btl * 8 + dr, pl.ds(v * L, L)] = (
                            plsc.load_gather(tab_v, [dsplats[dr], m16s[v]]))

            pltpu.async_copy(
                rows, out_sup.at[pl.ds(row0 + half * 128, 128)], sem_w)

    @pl.loop(0, NU // 2)
    def _pair(jp):
        for par in range(2):
            one_unit(jp * 2 + par, par)

    one_unit(NU - 1, 0)

    for rows in rows_bufs:
        pltpu.make_async_copy(rows, out_sup.at[pl.ds(0, 128)], sem_w).wait()


def _tc_query_body(qpat_ref, out_ref):
    out_ref[...] = qpat_ref[...]


_tc_query = pl.pallas_call(
    _tc_query_body,
    grid=(N,),
    in_specs=[pl.BlockSpec((NDT * NBT * 8, 128), lambda i: (0, 0))],
    out_specs=pl.BlockSpec((NDT * NBT * 8, 128), lambda i: (i, 0)),
    out_shape=jax.ShapeDtypeStruct((OUT_ROWS, 128), jnp.float32),
)


def kernel(y_support, padding_obs_support, n_obs_query, y_embedding, y_mask):
    yt = y_support.astype(jnp.int32).T.reshape(N, NBT, 128)
    padt = padding_obs_support.astype(jnp.int32).T.reshape(N, NBT, 128)
    tab_t = jnp.concatenate(
        [y_embedding,
         jnp.zeros((N_PAD - N_CLASSES, DIM), jnp.float32)], axis=0).T
    sup2d = _sc_embed(yt, padt, tab_t)
    qpat = jnp.broadcast_to(
        y_mask.reshape(NDT, 1, 8, 1), (NDT, NBT, 8, 128)).reshape(1024, 128)
    q2d = _tc_query(qpat)

    def to_out(x):
        x = x.reshape(N, NDT, NBT, 8, 128)
        x = x.transpose(2, 4, 0, 1, 3)
        return x.reshape(B, N, 1, DIM)

    return to_out(sup2d), to_out(q2d)
